# Initial kernel scaffold; baseline (speedup 1.0000x reference)
import jax
import jax.numpy as jnp
from jax.experimental import pallas as pl


def kernel(indices, codes, codebook):
    raise NotImplementedError("write your pallas kernel here")


# speedup vs baseline: 28.3873x; 28.3873x over previous
import functools

import jax
import jax.numpy as jnp
import numpy as np
from jax import lax
from jax.experimental import pallas as pl
from jax.experimental.pallas import tpu as pltpu
from jax.experimental.pallas import tpu_sc as plsc

DIM = 64
BLOCK_LEN = 4
NUM_BLOCKS = DIM // BLOCK_LEN
LANES = 16
K = 1024

NUM_CORES = 2
NUM_SUBCORES = 16
NW = NUM_CORES * NUM_SUBCORES

CHUNK = 800

_SEL = np.arange(LANES, dtype=np.int32) // BLOCK_LEN
_PAT = np.arange(LANES, dtype=np.int32) % BLOCK_LEN


@functools.partial(jax.jit, static_argnames=("n_chunks",))
def _decode(idx_flat, codes, cb_flat, n_chunks):
    n = idx_flat.shape[0]
    mesh = plsc.VectorSubcoreMesh(core_axis_name="c", subcore_axis_name="s")

    @functools.partial(
        pl.kernel,
        out_type=jax.ShapeDtypeStruct((n, DIM), jnp.float32),
        mesh=mesh,
        compiler_params=pltpu.CompilerParams(
            use_tc_tiling_on_sc=False, needs_layout_passes=False
        ),
        scratch_types=[
            pltpu.VMEM((K * BLOCK_LEN,), jnp.float32),
            pltpu.VMEM((CHUNK,), jnp.int32),
            pltpu.VMEM((CHUNK, NUM_BLOCKS), jnp.int32),
            pltpu.VMEM((CHUNK, DIM), jnp.float32),
            pltpu.SemaphoreType.DMA,
        ],
    )
    def k(idx_hbm, codes_hbm, cb_hbm, out_hbm, cb_v, idx_v, codes_v, dense_v,
          sem):
        wid = lax.axis_index("s") * NUM_CORES + lax.axis_index("c")
        per_w = n // NW
        lanes = lax.iota(jnp.int32, LANES)
        sel = lanes >> 2
        pat = lanes & 3
        pltpu.sync_copy(cb_hbm, cb_v)

        @pl.loop(0, n_chunks)
        def chunk(ci):
            base = wid * per_w + ci * CHUNK
            pltpu.sync_copy(idx_hbm.at[pl.ds(base, CHUNK)], idx_v)
            pltpu.async_copy(codes_hbm.at[idx_v], codes_v, sem).wait()

            @pl.loop(0, CHUNK)
            def decode(i):
                row = jnp.full((LANES,), i, dtype=jnp.int32)
                for v in range(DIM // LANES):
                    rid = plsc.load_gather(codes_v, [row, sel + v * BLOCK_LEN])
                    fid = (rid << 2) + pat
                    vals = plsc.load_gather(cb_v, [fid])
                    dense_v[i, pl.ds(v * LANES, LANES)] = vals

            pltpu.sync_copy(dense_v, out_hbm.at[pl.ds(base, CHUNK)])

    return k(idx_flat, codes, cb_flat)


def kernel(indices, codes, codebook):
    b, l = indices.shape
    n = b * l
    assert n % (NW * CHUNK) == 0
    out = _decode(
        indices.reshape(n), codes, codebook.reshape(K * BLOCK_LEN),
        n // (NW * CHUNK),
    )
    return out.reshape(b, l, DIM)


# speedup vs baseline: 67.5532x; 2.3797x over previous
import functools

import jax
import jax.numpy as jnp
from jax import lax
from jax.experimental import pallas as pl
from jax.experimental.pallas import tpu as pltpu
from jax.experimental.pallas import tpu_sc as plsc

DIM = 64
BLOCK_LEN = 4
NUM_BLOCKS = DIM // BLOCK_LEN
LANES = 16
K = 1024

NUM_CORES = 2
NUM_SUBCORES = 16
NW = NUM_CORES * NUM_SUBCORES

CHUNK = 640


@functools.partial(jax.jit, static_argnames=("n_chunks",))
def _decode(idx_flat, codes, cb_flat, n_chunks):
    n = idx_flat.shape[0]
    mesh = plsc.VectorSubcoreMesh(core_axis_name="c", subcore_axis_name="s")

    @functools.partial(
        pl.kernel,
        out_type=jax.ShapeDtypeStruct((n * DIM,), jnp.float32),
        mesh=mesh,
        compiler_params=pltpu.CompilerParams(
            use_tc_tiling_on_sc=False, needs_layout_passes=False
        ),
        scratch_types=[
            pltpu.VMEM((K * BLOCK_LEN,), jnp.float32),
            [pltpu.VMEM((CHUNK,), jnp.int32)] * 2,
            [pltpu.VMEM((CHUNK, NUM_BLOCKS), jnp.int32)] * 2,
            [pltpu.VMEM((CHUNK * DIM,), jnp.float32)] * 2,
            [pltpu.SemaphoreType.DMA] * 2,
            [pltpu.SemaphoreType.DMA] * 2,
        ],
    )
    def k(idx_hbm, codes_hbm, cb_hbm, out_hbm, cb_v, idx_v, codes_v, dense_v,
          gsem, osem):
        wid = lax.axis_index("s") * NUM_CORES + lax.axis_index("c")
        per_w = n // NW
        base0 = wid * per_w
        lanes = lax.iota(jnp.int32, LANES)
        sel = lanes >> 2
        pat = lanes & 3
        pltpu.sync_copy(cb_hbm, cb_v)

        def fetch(ci, b):
            pltpu.sync_copy(
                idx_hbm.at[pl.ds(base0 + ci * CHUNK, CHUNK)], idx_v[b]
            )
            pltpu.async_copy(codes_hbm.at[idx_v[b]], codes_v[b], gsem[b])

        fetch(0, 0)

        @pl.loop(0, n_chunks, step=2)
        def outer(ci0):
            for b in (0, 1):
                ci = ci0 + b

                @pl.when(ci + 1 < n_chunks)
                def _():
                    fetch(ci + 1, 1 - b)

                pltpu.make_async_copy(
                    codes_hbm.at[idx_v[b]], codes_v[b], gsem[b]
                ).wait()

                @pl.when(ci >= 2)
                def _():
                    pltpu.make_async_copy(
                        dense_v[b],
                        out_hbm.at[pl.ds(base0 * DIM, CHUNK * DIM)],
                        osem[b],
                    ).wait()

                cvb = codes_v[b]
                dvb = dense_v[b]

                @functools.partial(plsc.parallel_loop, 0, CHUNK, unroll=2)
                def decode(i):
                    row = jnp.full((LANES,), i, dtype=jnp.int32)
                    for v in range(DIM // LANES):
                        rid = plsc.load_gather(
                            cvb, [row, sel + v * BLOCK_LEN]
                        )
                        fid = (rid << 2) + pat
                        vals = plsc.load_gather(cb_v, [fid])
                        dvb[pl.ds(i * DIM + v * LANES, LANES)] = vals

                pltpu.async_copy(
                    dense_v[b],
                    out_hbm.at[pl.ds((base0 + ci * CHUNK) * DIM, CHUNK * DIM)],
                    osem[b],
                )

        for b in (0, 1):
            pltpu.make_async_copy(
                dense_v[b],
                out_hbm.at[pl.ds(base0 * DIM, CHUNK * DIM)],
                osem[b],
            ).wait()

    return k(idx_flat, codes, cb_flat)


def kernel(indices, codes, codebook):
    b, l = indices.shape
    n = b * l
    assert n % (NW * CHUNK) == 0 and (n // (NW * CHUNK)) % 2 == 0
    out = _decode(
        indices.reshape(n), codes, codebook.reshape(K * BLOCK_LEN),
        n // (NW * CHUNK),
    )
    return out.reshape(b, l, DIM)
